# baseline (speedup 1.0000x reference)
import functools

import jax
import jax.numpy as jnp
from jax import lax
from jax.experimental import pallas as pl
from jax.experimental.pallas import tpu as pltpu
from jax.experimental.pallas import tpu_sc as plsc

_B, _S, _D = 2, 2048, 1024
_E = 8
_FFN = 4096
_HALF = _FFN // 2
_T = _B * _S
_P = 2 * _T
_BLK = 128
_NBLK = _P // _BLK + _E
_R = _NBLK * _BLK
_NW = 32
_LANES = 16


def _router_body(x_ref, gw_ref, eid_ref, wgt_ref):
    xb = x_ref[...]
    logits = jax.lax.dot_general(
        xb, gw_ref[...], (((1,), (1,)), ((), ())),
        preferred_element_type=jnp.float32)
    probs = jax.nn.softmax(logits, axis=-1)
    i1 = jnp.argmax(probs, axis=-1)
    p1 = jnp.max(probs, axis=-1)
    cols = jax.lax.broadcasted_iota(jnp.int32, probs.shape, 1)
    masked = jnp.where(cols == i1[:, None], -1e30, probs)
    i2 = jnp.argmax(masked, axis=-1)
    p2 = jnp.max(masked, axis=-1)
    denom = p1 + p2
    eid_ref[0:1, :] = i1.astype(jnp.int32).reshape(1, _T)
    eid_ref[1:2, :] = i2.astype(jnp.int32).reshape(1, _T)
    wgt_ref[0:1, :] = (p1 / denom).reshape(1, _T)
    wgt_ref[1:2, :] = (p2 / denom).reshape(1, _T)


def _router(xf, gate_w):
    return pl.pallas_call(
        _router_body,
        grid=(1,),
        in_specs=[
            pl.BlockSpec((_T, _D), lambda i: (0, 0)),
            pl.BlockSpec((_E, _D), lambda i: (0, 0)),
        ],
        out_specs=[
            pl.BlockSpec((2, _T), lambda i: (0, 0)),
            pl.BlockSpec((2, _T), lambda i: (0, 0)),
        ],
        out_shape=[
            jax.ShapeDtypeStruct((2, _T), jnp.int32),
            jax.ShapeDtypeStruct((2, _T), jnp.float32),
        ],
    )(xf, gate_w)


_D_CH = 64
_D_PERW = _P // _NW
_D_NCH = _D_PERW // _D_CH


def _sc_dispatch(xf, pos):
    mesh = plsc.VectorSubcoreMesh(core_axis_name="c", subcore_axis_name="s")

    @functools.partial(
        pl.kernel, mesh=mesh,
        out_type=jax.ShapeDtypeStruct((_R, _D), jnp.float32),
        scratch_types=[
            pltpu.VMEM((_D_CH,), jnp.int32),
            pltpu.VMEM((_D_CH, _D), jnp.float32),
            pltpu.SemaphoreType.DMA,
        ],
    )
    def k(xf_hbm, pos_hbm, xs_hbm, idx_v, rows_v, sem):
        wid = lax.axis_index("s") * 2 + lax.axis_index("c")
        pbase = wid * _D_PERW
        sbase = pbase % _T
        for c in range(_D_NCH):
            pltpu.sync_copy(pos_hbm.at[pl.ds(pbase + c * _D_CH, _D_CH)], idx_v)
            pltpu.sync_copy(xf_hbm.at[pl.ds(sbase + c * _D_CH, _D_CH)], rows_v)
            pltpu.async_copy(rows_v, xs_hbm.at[idx_v], sem).wait()

    return k(xf, pos)


def _block_eid(b, off_ref):
    bb = b * _BLK
    s = 0
    for e in range(1, _E):
        s = s + (bb >= off_ref[e]).astype(jnp.int32)
    return s


def _ffn_body(off_ref, xs_ref, wup_ref, wdn_ref, ys_ref):
    xb = xs_ref[...].astype(jnp.bfloat16)
    h = jax.lax.dot_general(
        xb, wup_ref[0], (((1,), (1,)), ((), ())),
        preferred_element_type=jnp.float32)
    g = h[:, :_HALF]
    u = h[:, _HALF:]
    act = (g * jax.nn.sigmoid(g) * u).astype(jnp.bfloat16)
    ys_ref[...] = jax.lax.dot_general(
        act, wdn_ref[0], (((1,), (1,)), ((), ())),
        preferred_element_type=jnp.float32)


def _grouped_ffn(off, xs, w_up16, w_down16):
    grid_spec = pltpu.PrefetchScalarGridSpec(
        num_scalar_prefetch=1,
        grid=(_NBLK,),
        in_specs=[
            pl.BlockSpec((_BLK, _D), lambda b, off: (b, 0)),
            pl.BlockSpec((1, _FFN, _D), lambda b, off: (_block_eid(b, off), 0, 0)),
            pl.BlockSpec((1, _D, _HALF), lambda b, off: (_block_eid(b, off), 0, 0)),
        ],
        out_specs=pl.BlockSpec((_BLK, _D), lambda b, off: (b, 0)),
    )
    return pl.pallas_call(
        _ffn_body,
        grid_spec=grid_spec,
        out_shape=jax.ShapeDtypeStruct((_R, _D), jnp.float32),
    )(off, xs, w_up16, w_down16)


_C_CH = 32
_C_PERW = _T // _NW
_C_NCH = _C_PERW // _C_CH


def _sc_combine(ys, pos, wflat):
    mesh = plsc.VectorSubcoreMesh(core_axis_name="c", subcore_axis_name="s")

    @functools.partial(
        pl.kernel, mesh=mesh,
        out_type=jax.ShapeDtypeStruct((_T, _D), jnp.float32),
        scratch_types=[
            pltpu.VMEM((_C_CH,), jnp.int32),
            pltpu.VMEM((_C_CH,), jnp.int32),
            pltpu.VMEM((_C_CH,), jnp.float32),
            pltpu.VMEM((_C_CH,), jnp.float32),
            pltpu.VMEM((_C_CH, _D), jnp.float32),
            pltpu.VMEM((_C_CH, _D), jnp.float32),
            pltpu.VMEM((_C_CH, _D), jnp.float32),
            pltpu.SemaphoreType.DMA,
        ],
    )
    def k(ys_hbm, pos_hbm, w_hbm, out_hbm,
          idx1_v, idx2_v, w1_v, w2_v, r1_v, r2_v, o_v, sem):
        wid = lax.axis_index("s") * 2 + lax.axis_index("c")
        tbase = wid * _C_PERW
        for c in range(_C_NCH):
            b = tbase + c * _C_CH
            pltpu.sync_copy(pos_hbm.at[pl.ds(b, _C_CH)], idx1_v)
            pltpu.sync_copy(pos_hbm.at[pl.ds(_T + b, _C_CH)], idx2_v)
            pltpu.sync_copy(w_hbm.at[pl.ds(b, _C_CH)], w1_v)
            pltpu.sync_copy(w_hbm.at[pl.ds(_T + b, _C_CH)], w2_v)
            cp1 = pltpu.async_copy(ys_hbm.at[idx1_v], r1_v, sem)
            cp2 = pltpu.async_copy(ys_hbm.at[idx2_v], r2_v, sem)
            cp1.wait()
            cp2.wait()

            w1s = []
            w2s = []
            for g in range(_C_CH // _LANES):
                wa = w1_v[pl.ds(g * _LANES, _LANES)]
                wb = w2_v[pl.ds(g * _LANES, _LANES)]
                for i in range(_LANES):
                    w1s.append(wa[i])
                    w2s.append(wb[i])

            def col(j, carry):
                sl = pl.ds(j * _LANES, _LANES)
                for i in range(_C_CH):
                    o_v[i, sl] = w1s[i] * r1_v[i, sl] + w2s[i] * r2_v[i, sl]
                return carry

            lax.fori_loop(0, _D // _LANES, col, 0)
            pltpu.sync_copy(o_v, out_hbm.at[pl.ds(b, _C_CH)])

    return k(ys, pos, wflat)


def kernel(x, gate_w, w_up, w_down):
    xf = x.reshape(_T, _D)
    w_up16 = w_up.astype(jnp.bfloat16)
    w_down16 = w_down.astype(jnp.bfloat16)

    eids, wgts = _router(xf, gate_w)

    eflat = jnp.concatenate([eids[0], eids[1]])
    wflat = jnp.concatenate([wgts[0], wgts[1]])
    onehot = (eflat[:, None] == jnp.arange(_E)[None, :]).astype(jnp.int32)
    ranks = jnp.cumsum(onehot, axis=0) - onehot
    counts = jnp.sum(onehot, axis=0)
    pc = ((counts + _BLK - 1) // _BLK) * _BLK
    off = jnp.concatenate(
        [jnp.zeros((1,), jnp.int32), jnp.cumsum(pc)[:-1].astype(jnp.int32)])
    pos = (off[eflat] + jnp.sum(ranks * onehot, axis=1)).astype(jnp.int32)

    xs = _sc_dispatch(xf, pos)
    ys = _grouped_ffn(off, xs, w_up16, w_down16)
    out = _sc_combine(ys, pos, wflat)
    return out.reshape(_B, _S, _D)


# speedup vs baseline: 1.4775x; 1.4775x over previous
import functools

import jax
import jax.numpy as jnp
from jax import lax
from jax.experimental import pallas as pl
from jax.experimental.pallas import tpu as pltpu
from jax.experimental.pallas import tpu_sc as plsc

_B, _S, _D = 2, 2048, 1024
_E = 8
_FFN = 4096
_HALF = _FFN // 2
_T = _B * _S
_P = 2 * _T
_BLK = 256
_NBLK = _P // _BLK + _E
_R = _NBLK * _BLK
_NW = 32
_LANES = 16


def _router_body(x_ref, gw_ref, eid_ref, wgt_ref):
    xb = x_ref[...]
    logits = jax.lax.dot_general(
        xb, gw_ref[...], (((1,), (1,)), ((), ())),
        preferred_element_type=jnp.float32)
    probs = jax.nn.softmax(logits, axis=-1)
    i1 = jnp.argmax(probs, axis=-1)
    p1 = jnp.max(probs, axis=-1)
    cols = jax.lax.broadcasted_iota(jnp.int32, probs.shape, 1)
    masked = jnp.where(cols == i1[:, None], -1e30, probs)
    i2 = jnp.argmax(masked, axis=-1)
    p2 = jnp.max(masked, axis=-1)
    denom = p1 + p2
    eid_ref[0:1, :] = i1.astype(jnp.int32).reshape(1, _T)
    eid_ref[1:2, :] = i2.astype(jnp.int32).reshape(1, _T)
    wgt_ref[0:1, :] = (p1 / denom).reshape(1, _T)
    wgt_ref[1:2, :] = (p2 / denom).reshape(1, _T)


def _router(xf, gate_w):
    return pl.pallas_call(
        _router_body,
        grid=(1,),
        in_specs=[
            pl.BlockSpec((_T, _D), lambda i: (0, 0)),
            pl.BlockSpec((_E, _D), lambda i: (0, 0)),
        ],
        out_specs=[
            pl.BlockSpec((2, _T), lambda i: (0, 0)),
            pl.BlockSpec((2, _T), lambda i: (0, 0)),
        ],
        out_shape=[
            jax.ShapeDtypeStruct((2, _T), jnp.int32),
            jax.ShapeDtypeStruct((2, _T), jnp.float32),
        ],
    )(xf, gate_w)


_PR, _PCOLS = 64, 128


def _index_body(eid_ref, pos_ref, off_ref):
    e2 = eid_ref[...].reshape(_PR, _PCOLS)
    r128 = jax.lax.broadcasted_iota(jnp.int32, (_PCOLS, _PCOLS), 0)
    c128 = jax.lax.broadcasted_iota(jnp.int32, (_PCOLS, _PCOLS), 1)
    ut = (r128 <= c128).astype(jnp.float32)
    r64 = jax.lax.broadcasted_iota(jnp.int32, (_PR, _PR), 0)
    c64 = jax.lax.broadcasted_iota(jnp.int32, (_PR, _PR), 1)
    ls = (r64 > c64).astype(jnp.float32)

    ranks = []
    counts = []
    for e in range(_E):
        oh = (e2 == e).astype(jnp.float32)
        pref = jax.lax.dot_general(
            oh, ut, (((1,), (0,)), ((), ())),
            preferred_element_type=jnp.float32)
        rs = pref[:, _PCOLS - 1:_PCOLS]
        rowpref = jax.lax.dot_general(
            ls, rs, (((1,), (0,)), ((), ())),
            preferred_element_type=jnp.float32)
        ranks.append(pref - oh + rowpref)
        counts.append((rowpref[_PR - 1, 0] + rs[_PR - 1, 0]).reshape(1, 1))

    cvec = jnp.concatenate(counts, axis=1)
    pvec = jnp.ceil(cvec * (1.0 / _BLK)) * float(_BLK)
    r8 = jax.lax.broadcasted_iota(jnp.int32, (_E, _E), 0)
    c8 = jax.lax.broadcasted_iota(jnp.int32, (_E, _E), 1)
    ls8 = (r8 < c8).astype(jnp.float32)
    ovec = jax.lax.dot_general(
        pvec, ls8, (((1,), (0,)), ((), ())),
        preferred_element_type=jnp.float32)

    pos = jnp.zeros((_PR, _PCOLS), jnp.float32)
    for e in range(_E):
        pos = pos + jnp.where(e2 == e, ovec[0, e] + ranks[e], 0.0)
    pos_ref[...] = pos.astype(jnp.int32)
    off_ref[...] = ovec.astype(jnp.int32)


def _index(eids):
    return pl.pallas_call(
        _index_body,
        grid=(1,),
        in_specs=[pl.BlockSpec((2, _T), lambda i: (0, 0))],
        out_specs=[
            pl.BlockSpec((_PR, _PCOLS), lambda i: (0, 0)),
            pl.BlockSpec((1, _E), lambda i: (0, 0)),
        ],
        out_shape=[
            jax.ShapeDtypeStruct((_PR, _PCOLS), jnp.int32),
            jax.ShapeDtypeStruct((1, _E), jnp.int32),
        ],
    )(eids)


_D_CH = 64
_D_PERW = _P // _NW
_D_NCH = _D_PERW // _D_CH


def _sc_dispatch(xf, pos):
    mesh = plsc.VectorSubcoreMesh(core_axis_name="c", subcore_axis_name="s")

    @functools.partial(
        pl.kernel, mesh=mesh,
        out_type=jax.ShapeDtypeStruct((_R, _D), jnp.float32),
        scratch_types=[
            pltpu.VMEM((_D_CH,), jnp.int32),
            pltpu.VMEM((_D_CH, _D), jnp.float32),
            pltpu.SemaphoreType.DMA,
        ],
    )
    def k(xf_hbm, pos_hbm, xs_hbm, idx_v, rows_v, sem):
        wid = lax.axis_index("s") * 2 + lax.axis_index("c")
        pbase = wid * _D_PERW
        sbase = pbase % _T
        for c in range(_D_NCH):
            pltpu.sync_copy(pos_hbm.at[pl.ds(pbase + c * _D_CH, _D_CH)], idx_v)
            pltpu.sync_copy(xf_hbm.at[pl.ds(sbase + c * _D_CH, _D_CH)], rows_v)
            pltpu.async_copy(rows_v, xs_hbm.at[idx_v], sem).wait()

    return k(xf, pos)


def _block_eid(b, off_ref):
    bb = b * _BLK
    s = 0
    for e in range(1, _E):
        s = s + (bb >= off_ref[e]).astype(jnp.int32)
    return s


def _ffn_body(off_ref, xs_ref, wup_ref, wdn_ref, ys_ref):
    xb = xs_ref[...].astype(jnp.bfloat16)
    h = jax.lax.dot_general(
        xb, wup_ref[0], (((1,), (1,)), ((), ())),
        preferred_element_type=jnp.float32)
    g = h[:, :_HALF]
    u = h[:, _HALF:]
    act = (g * jax.nn.sigmoid(g) * u).astype(jnp.bfloat16)
    ys_ref[...] = jax.lax.dot_general(
        act, wdn_ref[0], (((1,), (1,)), ((), ())),
        preferred_element_type=jnp.float32)


def _grouped_ffn(off, xs, w_up16, w_down16):
    grid_spec = pltpu.PrefetchScalarGridSpec(
        num_scalar_prefetch=1,
        grid=(_NBLK,),
        in_specs=[
            pl.BlockSpec((_BLK, _D), lambda b, off: (b, 0)),
            pl.BlockSpec((1, _FFN, _D), lambda b, off: (_block_eid(b, off), 0, 0)),
            pl.BlockSpec((1, _D, _HALF), lambda b, off: (_block_eid(b, off), 0, 0)),
        ],
        out_specs=pl.BlockSpec((_BLK, _D), lambda b, off: (b, 0)),
    )
    return pl.pallas_call(
        _ffn_body,
        grid_spec=grid_spec,
        out_shape=jax.ShapeDtypeStruct((_R, _D), jnp.float32),
    )(off, xs, w_up16, w_down16)


_C_CH = 32
_C_PERW = _T // _NW
_C_NCH = _C_PERW // _C_CH


def _sc_combine(ys, pos, wflat):
    mesh = plsc.VectorSubcoreMesh(core_axis_name="c", subcore_axis_name="s")

    @functools.partial(
        pl.kernel, mesh=mesh,
        out_type=jax.ShapeDtypeStruct((_T, _D), jnp.float32),
        scratch_types=[
            pltpu.VMEM((_C_CH,), jnp.int32),
            pltpu.VMEM((_C_CH,), jnp.int32),
            pltpu.VMEM((_C_CH,), jnp.float32),
            pltpu.VMEM((_C_CH,), jnp.float32),
            pltpu.VMEM((_C_CH, _D), jnp.float32),
            pltpu.VMEM((_C_CH, _D), jnp.float32),
            pltpu.VMEM((_C_CH, _D), jnp.float32),
            pltpu.SemaphoreType.DMA,
        ],
    )
    def k(ys_hbm, pos_hbm, w_hbm, out_hbm,
          idx1_v, idx2_v, w1_v, w2_v, r1_v, r2_v, o_v, sem):
        wid = lax.axis_index("s") * 2 + lax.axis_index("c")
        tbase = wid * _C_PERW
        for c in range(_C_NCH):
            b = tbase + c * _C_CH
            pltpu.sync_copy(pos_hbm.at[pl.ds(b, _C_CH)], idx1_v)
            pltpu.sync_copy(pos_hbm.at[pl.ds(_T + b, _C_CH)], idx2_v)
            pltpu.sync_copy(w_hbm.at[pl.ds(b, _C_CH)], w1_v)
            pltpu.sync_copy(w_hbm.at[pl.ds(_T + b, _C_CH)], w2_v)
            cp1 = pltpu.async_copy(ys_hbm.at[idx1_v], r1_v, sem)
            cp2 = pltpu.async_copy(ys_hbm.at[idx2_v], r2_v, sem)
            cp1.wait()
            cp2.wait()

            w1s = []
            w2s = []
            for g in range(_C_CH // _LANES):
                wa = w1_v[pl.ds(g * _LANES, _LANES)]
                wb = w2_v[pl.ds(g * _LANES, _LANES)]
                for i in range(_LANES):
                    w1s.append(wa[i])
                    w2s.append(wb[i])

            def col(j, carry):
                sl = pl.ds(j * _LANES, _LANES)
                for i in range(_C_CH):
                    o_v[i, sl] = w1s[i] * r1_v[i, sl] + w2s[i] * r2_v[i, sl]
                return carry

            lax.fori_loop(0, _D // _LANES, col, 0)
            pltpu.sync_copy(o_v, out_hbm.at[pl.ds(b, _C_CH)])

    return k(ys, pos, wflat)


def kernel(x, gate_w, w_up, w_down):
    xf = x.reshape(_T, _D)
    w_up16 = w_up.astype(jnp.bfloat16)
    w_down16 = w_down.astype(jnp.bfloat16)

    eids, wgts = _router(xf, gate_w)
    pos2, off2 = _index(eids)
    pos = pos2.reshape(_P)
    off = off2.reshape(_E)
    wflat = wgts.reshape(_P)

    xs = _sc_dispatch(xf, pos)
    ys = _grouped_ffn(off, xs, w_up16, w_down16)
    out = _sc_combine(ys, pos, wflat)
    return out.reshape(_B, _S, _D)


# speedup vs baseline: 1.4786x; 1.0008x over previous
import functools

import jax
import jax.numpy as jnp
from jax import lax
from jax.experimental import pallas as pl
from jax.experimental.pallas import tpu as pltpu
from jax.experimental.pallas import tpu_sc as plsc

_B, _S, _D = 2, 2048, 1024
_E = 8
_FFN = 4096
_HALF = _FFN // 2
_T = _B * _S
_P = 2 * _T
_BLK = 256
_NBLK = _P // _BLK + _E
_R = _NBLK * _BLK
_NW = 32
_LANES = 16


def _router_body(x_ref, gw_ref, eid_ref, wgt_ref):
    xb = x_ref[...]
    logits = jax.lax.dot_general(
        xb, gw_ref[...], (((1,), (1,)), ((), ())),
        preferred_element_type=jnp.float32)
    probs = jax.nn.softmax(logits, axis=-1)
    i1 = jnp.argmax(probs, axis=-1)
    p1 = jnp.max(probs, axis=-1)
    cols = jax.lax.broadcasted_iota(jnp.int32, probs.shape, 1)
    masked = jnp.where(cols == i1[:, None], -1e30, probs)
    i2 = jnp.argmax(masked, axis=-1)
    p2 = jnp.max(masked, axis=-1)
    denom = p1 + p2
    eid_ref[0:1, :] = i1.astype(jnp.int32).reshape(1, _T)
    eid_ref[1:2, :] = i2.astype(jnp.int32).reshape(1, _T)
    wgt_ref[0:1, :] = (p1 / denom).reshape(1, _T)
    wgt_ref[1:2, :] = (p2 / denom).reshape(1, _T)


def _router(xf, gate_w):
    return pl.pallas_call(
        _router_body,
        grid=(1,),
        in_specs=[
            pl.BlockSpec((_T, _D), lambda i: (0, 0)),
            pl.BlockSpec((_E, _D), lambda i: (0, 0)),
        ],
        out_specs=[
            pl.BlockSpec((2, _T), lambda i: (0, 0)),
            pl.BlockSpec((2, _T), lambda i: (0, 0)),
        ],
        out_shape=[
            jax.ShapeDtypeStruct((2, _T), jnp.int32),
            jax.ShapeDtypeStruct((2, _T), jnp.float32),
        ],
    )(xf, gate_w)


_PR, _PCOLS = 64, 128


def _index_body(eid_ref, pos_ref, off_ref):
    e2 = eid_ref[...].reshape(_PR, _PCOLS)
    r128 = jax.lax.broadcasted_iota(jnp.int32, (_PCOLS, _PCOLS), 0)
    c128 = jax.lax.broadcasted_iota(jnp.int32, (_PCOLS, _PCOLS), 1)
    ut = (r128 <= c128).astype(jnp.float32)
    r64 = jax.lax.broadcasted_iota(jnp.int32, (_PR, _PR), 0)
    c64 = jax.lax.broadcasted_iota(jnp.int32, (_PR, _PR), 1)
    ls = (r64 > c64).astype(jnp.float32)

    ranks = []
    counts = []
    for e in range(_E):
        oh = (e2 == e).astype(jnp.float32)
        pref = jax.lax.dot_general(
            oh, ut, (((1,), (0,)), ((), ())),
            preferred_element_type=jnp.float32)
        rs = pref[:, _PCOLS - 1:_PCOLS]
        rowpref = jax.lax.dot_general(
            ls, rs, (((1,), (0,)), ((), ())),
            preferred_element_type=jnp.float32)
        ranks.append(pref - oh + rowpref)
        counts.append((rowpref[_PR - 1, 0] + rs[_PR - 1, 0]).reshape(1, 1))

    cvec = jnp.concatenate(counts, axis=1)
    pvec = jnp.ceil(cvec * (1.0 / _BLK)) * float(_BLK)
    r8 = jax.lax.broadcasted_iota(jnp.int32, (_E, _E), 0)
    c8 = jax.lax.broadcasted_iota(jnp.int32, (_E, _E), 1)
    ls8 = (r8 < c8).astype(jnp.float32)
    ovec = jax.lax.dot_general(
        pvec, ls8, (((1,), (0,)), ((), ())),
        preferred_element_type=jnp.float32)

    pos = jnp.zeros((_PR, _PCOLS), jnp.float32)
    for e in range(_E):
        pos = pos + jnp.where(e2 == e, ovec[0, e] + ranks[e], 0.0)
    pos_ref[...] = pos.astype(jnp.int32)
    off_ref[...] = ovec.astype(jnp.int32)


def _index(eids):
    return pl.pallas_call(
        _index_body,
        grid=(1,),
        in_specs=[pl.BlockSpec((2, _T), lambda i: (0, 0))],
        out_specs=[
            pl.BlockSpec((_PR, _PCOLS), lambda i: (0, 0)),
            pl.BlockSpec((1, _E), lambda i: (0, 0)),
        ],
        out_shape=[
            jax.ShapeDtypeStruct((_PR, _PCOLS), jnp.int32),
            jax.ShapeDtypeStruct((1, _E), jnp.int32),
        ],
    )(eids)


_D_CH = 32
_D_PERW = _P // _NW
_D_NCH = _D_PERW // _D_CH


def _sc_dispatch(xf, pos):
    mesh = plsc.VectorSubcoreMesh(core_axis_name="c", subcore_axis_name="s")

    @functools.partial(
        pl.kernel, mesh=mesh,
        out_type=jax.ShapeDtypeStruct((_R, _D), jnp.float32),
        scratch_types=[
            pltpu.VMEM((_D_CH,), jnp.int32),
            pltpu.VMEM((_D_CH,), jnp.int32),
            pltpu.VMEM((_D_CH, _D), jnp.float32),
            pltpu.VMEM((_D_CH, _D), jnp.float32),
            pltpu.SemaphoreType.DMA,
            pltpu.SemaphoreType.DMA,
            pltpu.SemaphoreType.DMA,
        ],
    )
    def k(xf_hbm, pos_hbm, xs_hbm, idx0, idx1, rows0, rows1, sem_in,
          sem_o0, sem_o1):
        wid = lax.axis_index("s") * 2 + lax.axis_index("c")
        pbase = wid * _D_PERW
        sbase = pbase % _T
        idx = (idx0, idx1)
        rows = (rows0, rows1)
        sem_o = (sem_o0, sem_o1)

        def fire_in(c):
            pltpu.sync_copy(pos_hbm.at[pl.ds(pbase + c * _D_CH, _D_CH)],
                            idx[c % 2])
            return pltpu.async_copy(
                xf_hbm.at[pl.ds(sbase + c * _D_CH, _D_CH)], rows[c % 2],
                sem_in)

        in_cp = fire_in(0)
        out_cp = [None, None]
        for c in range(_D_NCH):
            in_cp.wait()
            cp = pltpu.async_copy(rows[c % 2], xs_hbm.at[idx[c % 2]],
                                  sem_o[c % 2])
            out_cp[c % 2] = cp
            if c + 1 < _D_NCH:
                if out_cp[(c + 1) % 2] is not None:
                    out_cp[(c + 1) % 2].wait()
                in_cp = fire_in(c + 1)
        out_cp[(_D_NCH - 2) % 2].wait()
        out_cp[(_D_NCH - 1) % 2].wait()

    return k(xf, pos)


def _block_eid(b, off_ref):
    bb = b * _BLK
    s = 0
    for e in range(1, _E):
        s = s + (bb >= off_ref[e]).astype(jnp.int32)
    return s


def _ffn_body(off_ref, xs_ref, wup_ref, wdn_ref, ys_ref):
    xb = xs_ref[...].astype(jnp.bfloat16)
    h = jax.lax.dot_general(
        xb, wup_ref[0], (((1,), (1,)), ((), ())),
        preferred_element_type=jnp.float32)
    g = h[:, :_HALF]
    u = h[:, _HALF:]
    act = (g * jax.nn.sigmoid(g) * u).astype(jnp.bfloat16)
    ys_ref[...] = jax.lax.dot_general(
        act, wdn_ref[0], (((1,), (1,)), ((), ())),
        preferred_element_type=jnp.float32)


def _grouped_ffn(off, xs, w_up16, w_down16):
    grid_spec = pltpu.PrefetchScalarGridSpec(
        num_scalar_prefetch=1,
        grid=(_NBLK,),
        in_specs=[
            pl.BlockSpec((_BLK, _D), lambda b, off: (b, 0)),
            pl.BlockSpec((1, _FFN, _D), lambda b, off: (_block_eid(b, off), 0, 0)),
            pl.BlockSpec((1, _D, _HALF), lambda b, off: (_block_eid(b, off), 0, 0)),
        ],
        out_specs=pl.BlockSpec((_BLK, _D), lambda b, off: (b, 0)),
    )
    return pl.pallas_call(
        _ffn_body,
        grid_spec=grid_spec,
        out_shape=jax.ShapeDtypeStruct((_R, _D), jnp.float32),
    )(off, xs, w_up16, w_down16)


_C_CH = 32
_C_PERW = _T // _NW
_C_NCH = _C_PERW // _C_CH


def _sc_combine(ys, pos, wflat):
    mesh = plsc.VectorSubcoreMesh(core_axis_name="c", subcore_axis_name="s")

    @functools.partial(
        pl.kernel, mesh=mesh,
        out_type=jax.ShapeDtypeStruct((_T, _D), jnp.float32),
        scratch_types=[
            pltpu.VMEM((_C_CH,), jnp.int32),
            pltpu.VMEM((_C_CH,), jnp.int32),
            pltpu.VMEM((_C_CH,), jnp.float32),
            pltpu.VMEM((_C_CH,), jnp.float32),
            pltpu.VMEM((_C_CH, _D), jnp.float32),
            pltpu.VMEM((_C_CH, _D), jnp.float32),
            pltpu.VMEM((_C_CH, _D), jnp.float32),
            pltpu.SemaphoreType.DMA,
        ],
    )
    def k(ys_hbm, pos_hbm, w_hbm, out_hbm,
          idx1_v, idx2_v, w1_v, w2_v, r1_v, r2_v, o_v, sem):
        wid = lax.axis_index("s") * 2 + lax.axis_index("c")
        tbase = wid * _C_PERW
        for c in range(_C_NCH):
            b = tbase + c * _C_CH
            pltpu.sync_copy(pos_hbm.at[pl.ds(b, _C_CH)], idx1_v)
            pltpu.sync_copy(pos_hbm.at[pl.ds(_T + b, _C_CH)], idx2_v)
            pltpu.sync_copy(w_hbm.at[pl.ds(b, _C_CH)], w1_v)
            pltpu.sync_copy(w_hbm.at[pl.ds(_T + b, _C_CH)], w2_v)
            cp1 = pltpu.async_copy(ys_hbm.at[idx1_v], r1_v, sem)
            cp2 = pltpu.async_copy(ys_hbm.at[idx2_v], r2_v, sem)
            cp1.wait()
            cp2.wait()

            w1s = []
            w2s = []
            for g in range(_C_CH // _LANES):
                wa = w1_v[pl.ds(g * _LANES, _LANES)]
                wb = w2_v[pl.ds(g * _LANES, _LANES)]
                for i in range(_LANES):
                    w1s.append(wa[i])
                    w2s.append(wb[i])

            def col(j, carry):
                sl = pl.ds(j * _LANES, _LANES)
                for i in range(_C_CH):
                    o_v[i, sl] = w1s[i] * r1_v[i, sl] + w2s[i] * r2_v[i, sl]
                return carry

            lax.fori_loop(0, _D // _LANES, col, 0)
            pltpu.sync_copy(o_v, out_hbm.at[pl.ds(b, _C_CH)])

    return k(ys, pos, wflat)


def kernel(x, gate_w, w_up, w_down):
    xf = x.reshape(_T, _D)
    w_up16 = w_up.astype(jnp.bfloat16)
    w_down16 = w_down.astype(jnp.bfloat16)

    eids, wgts = _router(xf, gate_w)
    pos2, off2 = _index(eids)
    pos = pos2.reshape(_P)
    off = off2.reshape(_E)
    wflat = wgts.reshape(_P)

    xs = _sc_dispatch(xf, pos)
    ys = _grouped_ffn(off, xs, w_up16, w_down16)
    out = _sc_combine(ys, pos, wflat)
    return out.reshape(_B, _S, _D)
